# Initial kernel scaffold; baseline (speedup 1.0000x reference)
import functools
import math

import jax
import jax.numpy as jnp
from jax import lax
from jax.experimental import pallas as pl
from jax.experimental.pallas import tpu as pltpu
from jax.experimental.pallas import tpu_sc as plsc

_NSPEC = 89
_DIM = 128
_SCALE = 1.0 / math.sqrt(89.0)

_NC = 2
_NS = 16
_NW = _NC * _NS

_N = 100000
_BPW = _N // _NW
_C = 125
_NCHUNK = _BPW // _C


def _scale_body(t_ref, o_ref):
    o_ref[...] = t_ref[...] * _SCALE


_scale_call = pl.pallas_call(
    _scale_body,
    out_shape=jax.ShapeDtypeStruct((_NSPEC, _DIM), jnp.float32),
)

_mesh = plsc.VectorSubcoreMesh(core_axis_name="c", subcore_axis_name="s")


@functools.partial(
    pl.kernel,
    out_type=jax.ShapeDtypeStruct((_N, _DIM), jnp.float32),
    mesh=_mesh,
    scratch_types=[
        pltpu.VMEM((_NCHUNK, _C), jnp.int32),
        pltpu.VMEM((_C, _DIM), jnp.float32),
        pltpu.SemaphoreType.DMA,
    ],
)
def _gather(idx_hbm, table_hbm, out_hbm, idx_v, rows_v, sem):
    wid = lax.axis_index("s") * _NC + lax.axis_index("c")
    base = wid * _BPW
    pltpu.sync_copy(idx_hbm.at[wid], idx_v)

    def body(ci, carry):
        pltpu.async_copy(table_hbm.at[idx_v.at[ci]], rows_v, sem).wait()
        pltpu.sync_copy(rows_v, out_hbm.at[pl.ds(base + ci * _C, _C)])
        return carry

    lax.fori_loop(0, _NCHUNK, body, 0)


def kernel(node_specie, embeddings):
    idx = node_specie.astype(jnp.int32).reshape(_NW, _NCHUNK, _C)
    scaled = _scale_call(embeddings)
    return _gather(idx, scaled)


# speedup vs baseline: 1.7358x; 1.7358x over previous
import functools
import math

import jax
import jax.numpy as jnp
from jax import lax
from jax.experimental import pallas as pl
from jax.experimental.pallas import tpu as pltpu
from jax.experimental.pallas import tpu_sc as plsc

_NSPEC = 89
_DIM = 128
_SCALE = 1.0 / math.sqrt(89.0)

_NC = 2
_NS = 16
_NW = _NC * _NS

_N = 100000
_C = 128
_NFULL = _N // _C
_TAIL = _N - _NFULL * _C
_TAIL_OFF = _NFULL * _C
_TAIL_WID = _NFULL % _NW


def _scale_body(t_ref, o_ref):
    o_ref[...] = t_ref[...] * _SCALE


_scale_call = pl.pallas_call(
    _scale_body,
    out_shape=jax.ShapeDtypeStruct((_NSPEC, _DIM), jnp.float32),
)

_mesh = plsc.VectorSubcoreMesh(core_axis_name="c", subcore_axis_name="s")


@functools.partial(
    pl.kernel,
    out_type=jax.ShapeDtypeStruct((_N, _DIM), jnp.float32),
    mesh=_mesh,
    scratch_types=[
        pltpu.VMEM((_C,), jnp.int32),
        pltpu.VMEM((_TAIL,), jnp.int32),
        pltpu.VMEM((_C, _DIM), jnp.float32),
        pltpu.SemaphoreType.DMA,
    ],
)
def _gather(idx_hbm, table_hbm, out_hbm, idx_c, idx_t, rows_v, sem):
    wid = lax.axis_index("s") * _NC + lax.axis_index("c")
    count = (_NFULL - 1 - wid) // _NW + 1

    def body(k, carry):
        off = (wid + k * _NW) * _C
        pltpu.sync_copy(idx_hbm.at[pl.ds(off, _C)], idx_c)
        pltpu.async_copy(table_hbm.at[idx_c], rows_v, sem).wait()
        pltpu.sync_copy(rows_v, out_hbm.at[pl.ds(off, _C)])
        return carry

    lax.fori_loop(0, count, body, 0)

    @pl.when(wid == _TAIL_WID)
    def _():
        pltpu.sync_copy(idx_hbm.at[pl.ds(_TAIL_OFF, _TAIL)], idx_t)
        pltpu.async_copy(
            table_hbm.at[idx_t], rows_v.at[pl.ds(0, _TAIL)], sem
        ).wait()
        pltpu.sync_copy(
            rows_v.at[pl.ds(0, _TAIL)], out_hbm.at[pl.ds(_TAIL_OFF, _TAIL)]
        )


def kernel(node_specie, embeddings):
    idx = node_specie.astype(jnp.int32)
    scaled = _scale_call(embeddings)
    return _gather(idx, scaled)
